# baseline (speedup 1.0000x reference)
import jax
import jax.numpy as jnp
from jax.experimental import pallas as pl

GROUPS = 1024
GSIZE = 64
N = 16384
RROWS = 128


def _fps_body(x_ref, c_ref):
    x0 = x_ref[0, 0]
    x1 = x_ref[0, 1]
    x2 = x_ref[0, 2]
    rows = jax.lax.broadcasted_iota(jnp.int32, (RROWS, 128), 0)
    cols = jax.lax.broadcasted_iota(jnp.int32, (RROWS, 128), 1)
    flat = rows * 128 + cols
    crows = jax.lax.broadcasted_iota(jnp.int32, (8, 128), 0)
    ccols = jax.lax.broadcasted_iota(jnp.int32, (8, 128), 1)

    def step(s, carry):
        dist, f, a0, a1, a2 = carry
        m = flat == f
        c0 = jnp.sum(jnp.where(m, x0, 0.0), keepdims=True)
        c1 = jnp.sum(jnp.where(m, x1, 0.0), keepdims=True)
        c2 = jnp.sum(jnp.where(m, x2, 0.0), keepdims=True)
        sm = (crows == s // 128) & (ccols == s % 128)
        a0 = jnp.where(sm, c0, a0)
        a1 = jnp.where(sm, c1, a1)
        a2 = jnp.where(sm, c2, a2)
        d0 = x0 - c0
        d1 = x1 - c1
        d2 = x2 - c2
        d = (d0 * d0 + d1 * d1) + d2 * d2
        dist = jnp.minimum(dist, d)
        v = jnp.max(dist, keepdims=True)
        f = jnp.min(jnp.where(dist == v, flat, N), keepdims=True)
        return dist, f, a0, a1, a2

    init = (
        jnp.full((RROWS, 128), 1e10, dtype=jnp.float32),
        jnp.zeros((1, 1), dtype=jnp.int32),
        jnp.zeros((8, 128), dtype=jnp.float32),
        jnp.zeros((8, 128), dtype=jnp.float32),
        jnp.zeros((8, 128), dtype=jnp.float32),
    )
    _, _, a0, a1, a2 = jax.lax.fori_loop(0, GROUPS, step, init)
    c_ref[0, 0] = a0
    c_ref[0, 1] = a1
    c_ref[0, 2] = a2


def _fps_centers(xyz):
    B = xyz.shape[0]
    xg = jnp.transpose(xyz, (0, 2, 1)).reshape(B, 3, RROWS, 128)
    cacc = pl.pallas_call(
        _fps_body,
        grid=(B,),
        in_specs=[pl.BlockSpec((1, 3, RROWS, 128), lambda b: (b, 0, 0, 0))],
        out_specs=pl.BlockSpec((1, 3, 8, 128), lambda b: (b, 0, 0, 0)),
        out_shape=jax.ShapeDtypeStruct((B, 3, 8, 128), jnp.float32),
    )(xg)
    return jnp.transpose(cacc.reshape(B, 3, GROUPS), (0, 2, 1))


CTILE = 128


def _knn_body(ct_ref, xt_ref, idx_ref):
    A = ct_ref[0]
    X = xt_ref[0]
    d = -2.0 * jnp.dot(A, X, preferred_element_type=jnp.float32)
    d = d + jnp.sum(A * A, axis=1, keepdims=True)
    d = d + jnp.sum(X * X, axis=0, keepdims=True)

    def step(t, carry):
        d, acc = carry
        cols = jax.lax.broadcasted_iota(jnp.int32, (CTILE, N), 1)
        v = jnp.min(d, axis=1, keepdims=True)
        w = jnp.min(jnp.where(d == v, cols, N), axis=1, keepdims=True)
        d = jnp.where(cols == w, 3.0e38, d)
        t64 = jax.lax.broadcasted_iota(jnp.int32, (CTILE, GSIZE), 1)
        acc = jnp.where(t64 == t, w, acc)
        return d, acc

    init = (d, jnp.zeros((CTILE, GSIZE), dtype=jnp.int32))
    _, acc = jax.lax.fori_loop(0, GSIZE, step, init)
    idx_ref[0] = acc


def _knn_idx(center, xyz):
    B = xyz.shape[0]
    xt = jnp.transpose(xyz, (0, 2, 1))
    grid = (B, GROUPS // CTILE)
    return pl.pallas_call(
        _knn_body,
        grid=grid,
        in_specs=[
            pl.BlockSpec((1, CTILE, 3), lambda b, g: (b, g, 0)),
            pl.BlockSpec((1, 3, N), lambda b, g: (b, 0, 0)),
        ],
        out_specs=pl.BlockSpec((1, CTILE, GSIZE), lambda b, g: (b, g, 0)),
        out_shape=jax.ShapeDtypeStruct((B, GROUPS, GSIZE), jnp.int32),
    )(center, xt)


def kernel(xyz):
    B, n, _ = xyz.shape
    center = _fps_centers(xyz)
    idx = _knn_idx(center, xyz)
    idx_base = (jnp.arange(B, dtype=idx.dtype) * n)[:, None, None]
    idx_full = (idx + idx_base).reshape(-1)
    flat = xyz.reshape(B * n, 3)
    neighborhood = flat[idx_full].reshape(B, GROUPS, GSIZE, 3)
    neighborhood = neighborhood - center[:, :, None, :]
    return (neighborhood, center)


# speedup vs baseline: 1.0710x; 1.0710x over previous
import functools

import jax
import jax.numpy as jnp
from jax import lax
from jax.experimental import pallas as pl
from jax.experimental.pallas import tpu as pltpu
from jax.experimental.pallas import tpu_sc as plsc

GROUPS = 1024
GSIZE = 64
N = 16384
RROWS = 128


def _fps_body(x_ref, c_ref):
    x0 = x_ref[0, 0]
    x1 = x_ref[0, 1]
    x2 = x_ref[0, 2]
    rows = jax.lax.broadcasted_iota(jnp.int32, (RROWS, 128), 0)
    cols = jax.lax.broadcasted_iota(jnp.int32, (RROWS, 128), 1)
    flat = rows * 128 + cols
    crows = jax.lax.broadcasted_iota(jnp.int32, (8, 128), 0)
    ccols = jax.lax.broadcasted_iota(jnp.int32, (8, 128), 1)

    def step(s, carry):
        dist, f, a0, a1, a2 = carry
        m = flat == f
        c0 = jnp.sum(jnp.where(m, x0, 0.0), keepdims=True)
        c1 = jnp.sum(jnp.where(m, x1, 0.0), keepdims=True)
        c2 = jnp.sum(jnp.where(m, x2, 0.0), keepdims=True)
        sm = (crows == s // 128) & (ccols == s % 128)
        a0 = jnp.where(sm, c0, a0)
        a1 = jnp.where(sm, c1, a1)
        a2 = jnp.where(sm, c2, a2)
        d0 = x0 - c0
        d1 = x1 - c1
        d2 = x2 - c2
        d = (d0 * d0 + d1 * d1) + d2 * d2
        dist = jnp.minimum(dist, d)
        v = jnp.max(dist, keepdims=True)
        f = jnp.min(jnp.where(dist == v, flat, N), keepdims=True)
        return dist, f, a0, a1, a2

    init = (
        jnp.full((RROWS, 128), 1e10, dtype=jnp.float32),
        jnp.zeros((1, 1), dtype=jnp.int32),
        jnp.zeros((8, 128), dtype=jnp.float32),
        jnp.zeros((8, 128), dtype=jnp.float32),
        jnp.zeros((8, 128), dtype=jnp.float32),
    )
    _, _, a0, a1, a2 = jax.lax.fori_loop(0, GROUPS, step, init)
    c_ref[0, 0] = a0
    c_ref[0, 1] = a1
    c_ref[0, 2] = a2


def _fps_centers(xyz):
    B = xyz.shape[0]
    xg = jnp.transpose(xyz, (0, 2, 1)).reshape(B, 3, RROWS, 128)
    cacc = pl.pallas_call(
        _fps_body,
        grid=(B,),
        in_specs=[pl.BlockSpec((1, 3, RROWS, 128), lambda b: (b, 0, 0, 0))],
        out_specs=pl.BlockSpec((1, 3, 8, 128), lambda b: (b, 0, 0, 0)),
        out_shape=jax.ShapeDtypeStruct((B, 3, 8, 128), jnp.float32),
    )(xg)
    return jnp.transpose(cacc.reshape(B, 3, GROUPS), (0, 2, 1))


CTILE = 128


def _knn_body(ct_ref, xt_ref, idx_ref):
    A = ct_ref[0]
    X = xt_ref[0]
    d = -2.0 * jnp.dot(A, X, preferred_element_type=jnp.float32)
    d = d + jnp.sum(A * A, axis=1, keepdims=True)
    d = d + jnp.sum(X * X, axis=0, keepdims=True)

    def step(t, carry):
        d, acc = carry
        cols = jax.lax.broadcasted_iota(jnp.int32, (CTILE, N), 1)
        v = jnp.min(d, axis=1, keepdims=True)
        w = jnp.min(jnp.where(d == v, cols, N), axis=1, keepdims=True)
        d = jnp.where(cols == w, 3.0e38, d)
        t64 = jax.lax.broadcasted_iota(jnp.int32, (CTILE, GSIZE), 1)
        acc = jnp.where(t64 == t, w, acc)
        return d, acc

    init = (d, jnp.zeros((CTILE, GSIZE), dtype=jnp.int32))
    _, acc = jax.lax.fori_loop(0, GSIZE, step, init)
    idx_ref[0] = acc


def _knn_idx(center, xyz):
    B = xyz.shape[0]
    xt = jnp.transpose(xyz, (0, 2, 1))
    grid = (B, GROUPS // CTILE)
    return pl.pallas_call(
        _knn_body,
        grid=grid,
        in_specs=[
            pl.BlockSpec((1, CTILE, 3), lambda b, g: (b, g, 0)),
            pl.BlockSpec((1, 3, N), lambda b, g: (b, 0, 0)),
        ],
        out_specs=pl.BlockSpec((1, CTILE, GSIZE), lambda b, g: (b, g, 0)),
        out_shape=jax.ShapeDtypeStruct((B, GROUPS, GSIZE), jnp.int32),
    )(center, xt)


NW = 32
CHUNK = 128


def _gather_sub(pts, cen, idx_full, gidx, n_rows):
    rows_per_w = n_rows // NW
    n_chunks = rows_per_w // CHUNK
    mesh = plsc.VectorSubcoreMesh(core_axis_name="c", subcore_axis_name="s")
    fdt = jax.ShapeDtypeStruct((n_rows,), jnp.float32)

    @functools.partial(
        pl.kernel,
        mesh=mesh,
        out_type=(fdt, fdt, fdt),
        scratch_types=[
            pltpu.VMEM((rows_per_w,), jnp.int32),
            pltpu.VMEM((rows_per_w,), jnp.int32),
            pltpu.VMEM((CHUNK,), jnp.float32),
            pltpu.VMEM((CHUNK,), jnp.float32),
            pltpu.VMEM((CHUNK,), jnp.float32),
            pltpu.VMEM((CHUNK,), jnp.float32),
            pltpu.VMEM((CHUNK,), jnp.float32),
            pltpu.VMEM((CHUNK,), jnp.float32),
            pltpu.SemaphoreType.DMA,
        ],
    )
    def gk(xs, ys, zs, cxs, cys, czs, idx_hbm, gidx_hbm,
           ox, oy, oz, idx_v, gidx_v, px, py, pz, qx, qy, qz, sem):
        wid = lax.axis_index("s") * 2 + lax.axis_index("c")
        base = wid * rows_per_w
        pltpu.sync_copy(idx_hbm.at[pl.ds(base, rows_per_w)], idx_v)
        pltpu.sync_copy(gidx_hbm.at[pl.ds(base, rows_per_w)], gidx_v)

        def chunk_body(ci, carry):
            off = ci * CHUNK
            iv = idx_v.at[pl.ds(off, CHUNK)]
            gv = gidx_v.at[pl.ds(off, CHUNK)]
            hs = [pltpu.async_copy(src.at[i], dst, sem)
                  for src, i, dst in ((xs, iv, px), (ys, iv, py), (zs, iv, pz),
                                      (cxs, gv, qx), (cys, gv, qy), (czs, gv, qz))]
            for h in hs:
                h.wait()
            for v in range(CHUNK // 16):
                s = pl.ds(v * 16, 16)
                px[s] = px[s] - qx[s]
                py[s] = py[s] - qy[s]
                pz[s] = pz[s] - qz[s]
            pltpu.sync_copy(px, ox.at[pl.ds(base + off, CHUNK)])
            pltpu.sync_copy(py, oy.at[pl.ds(base + off, CHUNK)])
            pltpu.sync_copy(pz, oz.at[pl.ds(base + off, CHUNK)])
            return carry

        lax.fori_loop(0, n_chunks, chunk_body, 0)

    return gk(pts[0], pts[1], pts[2], cen[0], cen[1], cen[2], idx_full, gidx)


def kernel(xyz):
    B, n, _ = xyz.shape
    center = _fps_centers(xyz)
    idx = _knn_idx(center, xyz)
    idx_base = (jnp.arange(B, dtype=idx.dtype) * n)[:, None, None]
    idx_full = (idx + idx_base).reshape(-1)
    n_rows = B * GROUPS * GSIZE
    gidx = (jnp.arange(n_rows, dtype=jnp.int32) // GSIZE)
    flat = xyz.reshape(B * n, 3)
    cflat = center.reshape(B * GROUPS, 3)
    ox, oy, oz = _gather_sub(
        (flat[:, 0], flat[:, 1], flat[:, 2]),
        (cflat[:, 0], cflat[:, 1], cflat[:, 2]),
        idx_full, gidx, n_rows)
    neighborhood = jnp.stack([ox, oy, oz], axis=-1).reshape(B, GROUPS, GSIZE, 3)
    return (neighborhood, center)
